# baseline (device time: 34540 ns/iter reference)
import jax
import jax.numpy as jnp
from jax import lax
from jax.experimental import pallas as pl
from jax.experimental.pallas import tpu as pltpu

N_DEV = 4


def kernel(x, w_mat):
    m, _ = x.shape
    _, n = w_mat.shape
    ch = m // N_DEV

    def body(x_ref, w_ref, out_ref, partial_ref, acc_ref, rs_buf,
             rs_send_sems, rs_recv_sems, ag_send_sems, ag_recv_sems):
        me = lax.axis_index("i")

        barrier = pltpu.get_barrier_semaphore()
        for j in range(N_DEV):
            @pl.when(j != me)
            def _(j=j):
                pl.semaphore_signal(
                    barrier, inc=1, device_id=(j,),
                    device_id_type=pl.DeviceIdType.MESH,
                )
        pl.semaphore_wait(barrier, N_DEV - 1)

        xb = x_ref[...].astype(jnp.bfloat16)
        wb = w_ref[...].astype(jnp.bfloat16)
        partial_ref[...] = jnp.dot(
            xb, wb, preferred_element_type=jnp.float32
        ).astype(jnp.bfloat16)

        for j in range(N_DEV):
            @pl.when(j != me)
            def _(j=j):
                pltpu.make_async_remote_copy(
                    src_ref=partial_ref.at[pl.ds(j * ch, ch), :],
                    dst_ref=rs_buf.at[me],
                    send_sem=rs_send_sems.at[j],
                    recv_sem=rs_recv_sems.at[me],
                    device_id=(j,),
                    device_id_type=pl.DeviceIdType.MESH,
                ).start()

        acc_ref[...] = partial_ref[pl.ds(me * ch, ch), :].astype(jnp.float32)

        for j in range(N_DEV):
            @pl.when(j != me)
            def _(j=j):
                pltpu.make_async_remote_copy(
                    src_ref=rs_buf.at[j],
                    dst_ref=rs_buf.at[j],
                    send_sem=rs_send_sems.at[j],
                    recv_sem=rs_recv_sems.at[j],
                    device_id=(j,),
                    device_id_type=pl.DeviceIdType.MESH,
                ).wait_recv()
                acc_ref[...] += rs_buf[j].astype(jnp.float32)

        acc = acc_ref[...]
        y = acc / (1.0 + jnp.exp(-jnp.clip(acc, -60.0, 60.0)))
        out_ref[pl.ds(me * ch, ch), :] = y.astype(jnp.bfloat16)

        for j in range(N_DEV):
            @pl.when(j != me)
            def _(j=j):
                pltpu.make_async_remote_copy(
                    src_ref=out_ref.at[pl.ds(me * ch, ch), :],
                    dst_ref=out_ref.at[pl.ds(me * ch, ch), :],
                    send_sem=ag_send_sems.at[j],
                    recv_sem=ag_recv_sems.at[me],
                    device_id=(j,),
                    device_id_type=pl.DeviceIdType.MESH,
                ).start()

        for j in range(N_DEV):
            @pl.when(j != me)
            def _(j=j):
                pltpu.make_async_remote_copy(
                    src_ref=partial_ref.at[pl.ds(j * ch, ch), :],
                    dst_ref=rs_buf.at[me],
                    send_sem=rs_send_sems.at[j],
                    recv_sem=rs_recv_sems.at[me],
                    device_id=(j,),
                    device_id_type=pl.DeviceIdType.MESH,
                ).wait_send()

        for j in range(N_DEV):
            @pl.when(j != me)
            def _(j=j):
                pltpu.make_async_remote_copy(
                    src_ref=out_ref.at[pl.ds(j * ch, ch), :],
                    dst_ref=out_ref.at[pl.ds(j * ch, ch), :],
                    send_sem=ag_send_sems.at[j],
                    recv_sem=ag_recv_sems.at[j],
                    device_id=(j,),
                    device_id_type=pl.DeviceIdType.MESH,
                ).wait_recv()

        for j in range(N_DEV):
            @pl.when(j != me)
            def _(j=j):
                pltpu.make_async_remote_copy(
                    src_ref=out_ref.at[pl.ds(me * ch, ch), :],
                    dst_ref=out_ref.at[pl.ds(me * ch, ch), :],
                    send_sem=ag_send_sems.at[j],
                    recv_sem=ag_recv_sems.at[j],
                    device_id=(j,),
                    device_id_type=pl.DeviceIdType.MESH,
                ).wait_send()

    return pl.pallas_call(
        body,
        out_shape=jax.ShapeDtypeStruct((m, n), jnp.bfloat16),
        in_specs=[
            pl.BlockSpec(memory_space=pltpu.VMEM),
            pl.BlockSpec(memory_space=pltpu.VMEM),
        ],
        out_specs=pl.BlockSpec(memory_space=pltpu.VMEM),
        scratch_shapes=[
            pltpu.VMEM((m, n), jnp.bfloat16),
            pltpu.VMEM((ch, n), jnp.float32),
            pltpu.VMEM((N_DEV, ch, n), jnp.bfloat16),
            pltpu.SemaphoreType.DMA((N_DEV,)),
            pltpu.SemaphoreType.DMA((N_DEV,)),
            pltpu.SemaphoreType.DMA((N_DEV,)),
            pltpu.SemaphoreType.DMA((N_DEV,)),
        ],
        compiler_params=pltpu.CompilerParams(collective_id=0),
    )(x, w_mat)
